# Initial kernel scaffold; baseline (speedup 1.0000x reference)
import jax
import jax.numpy as jnp
from jax.experimental import pallas as pl


def kernel(input, target, weight, bias, unigram_prob):
    raise NotImplementedError("write your pallas kernel here")


# speedup vs baseline: 2.6575x; 2.6575x over previous
import functools

import jax
import jax.numpy as jnp
from jax import lax
from jax.experimental import pallas as pl
from jax.experimental.pallas import tpu as pltpu
from jax.experimental.pallas import tpu_sc as plsc

N = 16384
IDIM = 128
ODIM = 1000
K = 64
KP1 = K + 1
CPAD = 1024

_info = plsc.get_sparse_core_info()
_NC, _NS, _L = _info.num_cores, _info.num_subcores, _info.num_lanes
_NW = _NC * _NS
_CHUNK = N // _NW
_STEP = 128
_NSTEPS = _CHUNK // _STEP


def _sc_body(x_hbm, tgt_hbm, w_hbm, cvec_hbm, noise_hbm,
             pmt_hbm, wn_hbm, cn_hbm,
             xv, wv, idxc, pmt_v, cv, nidx, wnv, cnv, sem):
    cid = lax.axis_index("c")
    sid = lax.axis_index("s")
    wid = sid * _NC + cid
    base = wid * _CHUNK

    lane = lax.iota(jnp.int32, _L)
    p1, p2, p4, p8 = lane ^ 1, lane ^ 2, lane ^ 4, lane ^ 8

    pltpu.sync_copy(cvec_hbm, cv)

    for step in range(_NSTEPS):
        rb = base + step * _STEP
        pltpu.sync_copy(tgt_hbm.at[pl.ds(rb, _STEP)], idxc)
        pltpu.sync_copy(x_hbm.at[pl.ds(rb, _STEP)], xv)
        pltpu.async_copy(w_hbm.at[idxc], wv, sem).wait()

        def group(gi, _, _step=step):
            res = jnp.zeros((_L,), jnp.float32)
            tg = idxc[pl.ds(gi * _L, _L)]
            for u in range(_L):
                r = gi * _L + u
                acc = xv[r, pl.ds(0, _L)] * wv[r, pl.ds(0, _L)]
                for q in range(1, IDIM // _L):
                    acc = acc + (xv[r, pl.ds(q * _L, _L)] *
                                 wv[r, pl.ds(q * _L, _L)])
                acc = acc + jnp.take(acc, p1)
                acc = acc + jnp.take(acc, p2)
                acc = acc + jnp.take(acc, p4)
                acc = acc + jnp.take(acc, p8)
                acc = acc + cv[pl.ds(tg[u], _L)][0]
                res = jnp.where(lane == u, acc, res)
            pmt_v[pl.ds(_step * _STEP + gi * _L, _L)] = res
            return 0

        lax.fori_loop(0, _STEP // _L, group, 0)

    pltpu.sync_copy(pmt_v, pmt_hbm.at[pl.ds(base, _CHUNK)])

    @pl.when(wid == 0)
    def _():
        pltpu.sync_copy(noise_hbm, nidx)
        pltpu.async_copy(w_hbm.at[nidx], wnv, sem).wait()
        pltpu.sync_copy(wnv, wn_hbm)
        for gq in range(K // _L):
            nv = nidx[pl.ds(gq * _L, _L)]
            resn = jnp.zeros((_L,), jnp.float32)
            for u in range(_L):
                resn = jnp.where(lane == u, cv[pl.ds(nv[u], _L)][0], resn)
            cnv[pl.ds(gq * _L, _L)] = resn
        pltpu.sync_copy(cnv, cn_hbm)


_sc_call = functools.partial(
    pl.kernel,
    mesh=plsc.VectorSubcoreMesh(core_axis_name="c", subcore_axis_name="s"),
    out_type=[
        jax.ShapeDtypeStruct((N,), jnp.float32),
        jax.ShapeDtypeStruct((K, IDIM), jnp.float32),
        jax.ShapeDtypeStruct((K,), jnp.float32),
    ],
    scratch_types=[
        pltpu.VMEM((_STEP, IDIM), jnp.float32),
        pltpu.VMEM((_STEP, IDIM), jnp.float32),
        pltpu.VMEM((_STEP,), jnp.int32),
        pltpu.VMEM((_CHUNK,), jnp.float32),
        pltpu.VMEM((CPAD,), jnp.float32),
        pltpu.VMEM((K,), jnp.int32),
        pltpu.VMEM((K, IDIM), jnp.float32),
        pltpu.VMEM((K,), jnp.float32),
        pltpu.SemaphoreType.DMA,
    ],
)(_sc_body)


_B = 1024


def _tc_body(x_ref, pmt_ref, wt_ref, cpad_ref, logits_ref, ones_ref):
    m = lax.dot_general(x_ref[...], wt_ref[...],
                        dimension_numbers=(((1,), (0,)), ((), ())),
                        preferred_element_type=jnp.float32)
    col = lax.broadcasted_iota(jnp.int32, (_B, KP1), 1)
    logits_ref[...] = jnp.where(col == 0, pmt_ref[...], -(m + cpad_ref[...]))
    ones_ref[...] = jnp.ones((_B, KP1), jnp.float32)


_tc_call = pl.pallas_call(
    _tc_body,
    grid=(N // _B,),
    in_specs=[
        pl.BlockSpec((_B, IDIM), lambda i: (i, 0)),
        pl.BlockSpec((_B, 1), lambda i: (i, 0)),
        pl.BlockSpec((IDIM, KP1), lambda i: (0, 0)),
        pl.BlockSpec((1, KP1), lambda i: (0, 0)),
    ],
    out_specs=[
        pl.BlockSpec((_B, KP1), lambda i: (i, 0)),
        pl.BlockSpec((_B, KP1), lambda i: (i, 0)),
    ],
    out_shape=[
        jax.ShapeDtypeStruct((N, KP1), jnp.float32),
        jax.ShapeDtypeStruct((N, KP1), jnp.float32),
    ],
)


def kernel(input, target, weight, bias, unigram_prob):
    noise = jax.random.randint(jax.random.key(42), (K,), 0, weight.shape[0])
    cvec = bias - jnp.log(K * unigram_prob)
    cvec_pad = jnp.zeros((CPAD,), jnp.float32).at[:ODIM].set(cvec)

    pmt, wn, cn = _sc_call(input, target.astype(jnp.int32), weight,
                           cvec_pad, noise.astype(jnp.int32))

    wt_pad = jnp.concatenate(
        [jnp.zeros((IDIM, 1), jnp.float32), wn.T], axis=1)
    cn_pad = jnp.concatenate(
        [jnp.zeros((1,), jnp.float32), cn]).reshape(1, KP1)

    logits, nce_target = _tc_call(input, pmt.reshape(N, 1), wt_pad, cn_pad)
    return (logits, nce_target)
